# baseline (speedup 1.0000x reference)
import functools
import math

import jax
import jax.numpy as jnp
import numpy as np
from jax import lax
from jax.experimental import pallas as pl
from jax.experimental.pallas import tpu as pltpu
from jax.experimental.pallas import tpu_sc as plsc

MAX_LOOPS = 100000
HIDDEN_DIM = 64
_B = 4096
_T = 200


def _make_pe_np(max_loops: int, hidden_dim: int) -> np.ndarray:
    position = np.arange(0, max_loops, dtype=np.float32)[:, None]
    div_term = np.exp(
        np.arange(0, hidden_dim, 2, dtype=np.float32)
        * (-math.log(10000.0) / hidden_dim)
    )
    pe = np.zeros((max_loops, hidden_dim), dtype=np.float32)
    pe[:, 0::2] = np.sin(position * div_term)
    pe[:, 1::2] = np.cos(position * div_term)
    return pe


_PE_T = np.ascontiguousarray(_make_pe_np(MAX_LOOPS, HIDDEN_DIM).T)

_A_BLK = 8192


def _add_body(t_ref, p_ref, o_ref):
    o_ref[...] = t_ref[...] + p_ref[...]


def _fuse_table_t(table_t):
    grid = (MAX_LOOPS + _A_BLK - 1) // _A_BLK
    return pl.pallas_call(
        _add_body,
        grid=(grid,),
        in_specs=[
            pl.BlockSpec((HIDDEN_DIM, _A_BLK), lambda i: (0, i)),
            pl.BlockSpec((HIDDEN_DIM, _A_BLK), lambda i: (0, i)),
        ],
        out_specs=pl.BlockSpec((HIDDEN_DIM, _A_BLK), lambda i: (0, i)),
        out_shape=jax.ShapeDtypeStruct((HIDDEN_DIM, MAX_LOOPS), jnp.float32),
    )(table_t, _PE_T)


_L = 16
_H_PER_W = 2


def _gather_t(fused_t, idx_t):
    mesh = plsc.VectorSubcoreMesh(core_axis_name="c", subcore_axis_name="s")

    @functools.partial(
        pl.kernel,
        out_type=jax.ShapeDtypeStruct((_T, HIDDEN_DIM, _B), jnp.float32),
        mesh=mesh,
        scratch_types=[
            pltpu.VMEM((MAX_LOOPS,), jnp.float32),
            pltpu.VMEM((_B,), jnp.int32),
            pltpu.VMEM((_B,), jnp.float32),
        ],
        compiler_params=pltpu.CompilerParams(
            use_tc_tiling_on_sc=True, needs_layout_passes=False
        ),
    )
    def k(fused_hbm, idx_hbm, out_hbm, frow, idxv, obuf):
        wid = lax.axis_index("s") * 2 + lax.axis_index("c")
        for j in range(_H_PER_W):
            h = wid * _H_PER_W + j
            pltpu.sync_copy(fused_hbm.at[h], frow)

            def tbody(t, carry):
                pltpu.sync_copy(idx_hbm.at[t], idxv)

                def gbody(g, c2):
                    vi = idxv[pl.ds(g * _L, _L)]
                    obuf[pl.ds(g * _L, _L)] = plsc.load_gather(frow, [vi])
                    return c2

                lax.fori_loop(0, _B // _L, gbody, 0)
                pltpu.sync_copy(obuf, out_hbm.at[t, h])
                return carry

            lax.fori_loop(0, _T, tbody, 0)

    return k(fused_t, idx_t)


def kernel(loop_idx, embedding_table):
    idx_t = jnp.minimum(loop_idx, MAX_LOOPS - 1).T
    table_t = embedding_table.T
    fused_t = _fuse_table_t(table_t)
    out_t = _gather_t(fused_t, idx_t)
    return jnp.transpose(out_t, (2, 0, 1))


# speedup vs baseline: 1.0086x; 1.0086x over previous
import functools
import math

import jax
import jax.numpy as jnp
import numpy as np
from jax import lax
from jax.experimental import pallas as pl
from jax.experimental.pallas import tpu as pltpu
from jax.experimental.pallas import tpu_sc as plsc

MAX_LOOPS = 100000
HIDDEN_DIM = 64
_B = 4096
_T = 200


def _make_pe_np(max_loops: int, hidden_dim: int) -> np.ndarray:
    position = np.arange(0, max_loops, dtype=np.float32)[:, None]
    div_term = np.exp(
        np.arange(0, hidden_dim, 2, dtype=np.float32)
        * (-math.log(10000.0) / hidden_dim)
    )
    pe = np.zeros((max_loops, hidden_dim), dtype=np.float32)
    pe[:, 0::2] = np.sin(position * div_term)
    pe[:, 1::2] = np.cos(position * div_term)
    return pe


_PE_T = np.ascontiguousarray(_make_pe_np(MAX_LOOPS, HIDDEN_DIM).T)

_A_BLK = 8192


def _add_body(t_ref, p_ref, o_ref):
    o_ref[...] = t_ref[...] + p_ref[...]


def _fuse_table_t(table_t):
    grid = (MAX_LOOPS + _A_BLK - 1) // _A_BLK
    return pl.pallas_call(
        _add_body,
        grid=(grid,),
        in_specs=[
            pl.BlockSpec((HIDDEN_DIM, _A_BLK), lambda i: (0, i)),
            pl.BlockSpec((HIDDEN_DIM, _A_BLK), lambda i: (0, i)),
        ],
        out_specs=pl.BlockSpec((HIDDEN_DIM, _A_BLK), lambda i: (0, i)),
        out_shape=jax.ShapeDtypeStruct((HIDDEN_DIM, MAX_LOOPS), jnp.float32),
    )(table_t, _PE_T)


_L = 16
_H_PER_W = 2


def _gather_t(fused_t, idx_t):
    mesh = plsc.VectorSubcoreMesh(core_axis_name="c", subcore_axis_name="s")

    _UNROLL = 8
    _G = _B // _L // _UNROLL

    @functools.partial(
        pl.kernel,
        out_type=jax.ShapeDtypeStruct((_T, HIDDEN_DIM, _B), jnp.float32),
        mesh=mesh,
        scratch_types=[
            pltpu.VMEM((MAX_LOOPS,), jnp.float32),
            pltpu.VMEM((2, _B), jnp.int32),
            pltpu.VMEM((2, _B), jnp.float32),
            pltpu.SemaphoreType.DMA((2,)),
            pltpu.SemaphoreType.DMA((2,)),
        ],
        compiler_params=pltpu.CompilerParams(
            use_tc_tiling_on_sc=True, needs_layout_passes=False
        ),
    )
    def k(fused_hbm, idx_hbm, out_hbm, frow, idxv, obuf, sin, sout):
        wid = lax.axis_index("s") * 2 + lax.axis_index("c")
        for j in range(_H_PER_W):
            h = wid * _H_PER_W + j
            pltpu.sync_copy(fused_hbm.at[h], frow)
            pltpu.async_copy(idx_hbm.at[0], idxv.at[0], sin.at[0])

            def tbody(t, carry):
                cur = lax.rem(t, 2)
                nxt = 1 - cur

                @pl.when(t + 1 < _T)
                def _():
                    pltpu.async_copy(idx_hbm.at[t + 1], idxv.at[nxt], sin.at[nxt])

                pltpu.make_async_copy(
                    idx_hbm.at[t], idxv.at[cur], sin.at[cur]
                ).wait()

                @pl.when(t >= 2)
                def _():
                    pltpu.make_async_copy(
                        obuf.at[cur], out_hbm.at[t - 2, h], sout.at[cur]
                    ).wait()

                def gbody(g, c2):
                    for u in range(_UNROLL):
                        o = (g * _UNROLL + u) * _L
                        vi = idxv[cur, pl.ds(o, _L)]
                        obuf[cur, pl.ds(o, _L)] = plsc.load_gather(frow, [vi])
                    return c2

                lax.fori_loop(0, _G, gbody, 0)
                pltpu.async_copy(obuf.at[cur], out_hbm.at[t, h], sout.at[cur])
                return carry

            lax.fori_loop(0, _T, tbody, 0)
            for tt in (_T - 2, _T - 1):
                pltpu.make_async_copy(
                    obuf.at[tt % 2], out_hbm.at[tt, h], sout.at[tt % 2]
                ).wait()

    return k(fused_t, idx_t)


def kernel(loop_idx, embedding_table):
    idx_t = jnp.minimum(loop_idx, MAX_LOOPS - 1).T
    table_t = embedding_table.T
    fused_t = _fuse_table_t(table_t)
    out_t = _gather_t(fused_t, idx_t)
    return jnp.transpose(out_t, (2, 0, 1))


# speedup vs baseline: 2.8285x; 2.8045x over previous
import functools
import math

import jax
import jax.numpy as jnp
import numpy as np
from jax import lax
from jax.experimental import pallas as pl
from jax.experimental.pallas import tpu as pltpu
from jax.experimental.pallas import tpu_sc as plsc

MAX_LOOPS = 100000
HIDDEN_DIM = 64
_B = 4096
_T = 200


def _make_pe_np(max_loops: int, hidden_dim: int) -> np.ndarray:
    position = np.arange(0, max_loops, dtype=np.float32)[:, None]
    div_term = np.exp(
        np.arange(0, hidden_dim, 2, dtype=np.float32)
        * (-math.log(10000.0) / hidden_dim)
    )
    pe = np.zeros((max_loops, hidden_dim), dtype=np.float32)
    pe[:, 0::2] = np.sin(position * div_term)
    pe[:, 1::2] = np.cos(position * div_term)
    return pe


_PE_T = np.ascontiguousarray(_make_pe_np(MAX_LOOPS, HIDDEN_DIM).T)

_A_BLK = 8192


def _add_body(t_ref, p_ref, o_ref):
    o_ref[...] = t_ref[...] + p_ref[...]


def _fuse_table_t(table_t):
    grid = (MAX_LOOPS + _A_BLK - 1) // _A_BLK
    return pl.pallas_call(
        _add_body,
        grid=(grid,),
        in_specs=[
            pl.BlockSpec((HIDDEN_DIM, _A_BLK), lambda i: (0, i)),
            pl.BlockSpec((HIDDEN_DIM, _A_BLK), lambda i: (0, i)),
        ],
        out_specs=pl.BlockSpec((HIDDEN_DIM, _A_BLK), lambda i: (0, i)),
        out_shape=jax.ShapeDtypeStruct((HIDDEN_DIM, MAX_LOOPS), jnp.float32),
    )(table_t, _PE_T)


_L = 16
_H_PER_W = 2


def _gather_t(fused_t, idx_t):
    mesh = plsc.VectorSubcoreMesh(core_axis_name="c", subcore_axis_name="s")

    _UNROLL = 8
    _G = _B // _L // _UNROLL

    @functools.partial(
        pl.kernel,
        out_type=jax.ShapeDtypeStruct((_T, HIDDEN_DIM, _B), jnp.float32),
        mesh=mesh,
        scratch_types=[
            pltpu.VMEM((MAX_LOOPS,), jnp.float32),
            pltpu.VMEM((2, _B), jnp.int32),
            pltpu.VMEM((2, _B), jnp.float32),
            pltpu.SemaphoreType.DMA((2,)),
            pltpu.SemaphoreType.DMA((2,)),
        ],
        compiler_params=pltpu.CompilerParams(
            use_tc_tiling_on_sc=True, needs_layout_passes=False
        ),
    )
    def k(fused_hbm, idx_hbm, out_hbm, frow, idxv, obuf, sin, sout):
        wid = lax.axis_index("s") * 2 + lax.axis_index("c")
        for j in range(_H_PER_W):
            h = wid * _H_PER_W + j
            pltpu.sync_copy(fused_hbm.at[h], frow)
            pltpu.async_copy(idx_hbm.at[0], idxv.at[0], sin.at[0])

            def tbody(t, carry):
                cur = lax.rem(t, 2)
                nxt = 1 - cur

                @pl.when(t + 1 < _T)
                def _():
                    pltpu.async_copy(idx_hbm.at[t + 1], idxv.at[nxt], sin.at[nxt])

                pltpu.make_async_copy(
                    idx_hbm.at[t], idxv.at[cur], sin.at[cur]
                ).wait()

                @pl.when(t >= 2)
                def _():
                    pltpu.make_async_copy(
                        obuf.at[cur], out_hbm.at[t - 2, h], sout.at[cur]
                    ).wait()

                @plsc.parallel_loop(0, _B // _L, unroll=_UNROLL)
                def _(g):
                    o = g * _L
                    vi = idxv[cur, pl.ds(o, _L)]
                    obuf[cur, pl.ds(o, _L)] = plsc.load_gather(frow, [vi])
                pltpu.async_copy(obuf.at[cur], out_hbm.at[t, h], sout.at[cur])
                return carry

            lax.fori_loop(0, _T, tbody, 0)
            for tt in (_T - 2, _T - 1):
                pltpu.make_async_copy(
                    obuf.at[tt % 2], out_hbm.at[tt, h], sout.at[tt % 2]
                ).wait()

    return k(fused_t, idx_t)


def kernel(loop_idx, embedding_table):
    idx_t = jnp.minimum(loop_idx, MAX_LOOPS - 1).T
    table_t = embedding_table.T
    fused_t = _fuse_table_t(table_t)
    out_t = _gather_t(fused_t, idx_t)
    return jnp.transpose(out_t, (2, 0, 1))


# speedup vs baseline: 4.3027x; 1.5212x over previous
import functools
import math

import jax
import jax.numpy as jnp
import numpy as np
from jax import lax
from jax.experimental import pallas as pl
from jax.experimental.pallas import tpu as pltpu
from jax.experimental.pallas import tpu_sc as plsc

MAX_LOOPS = 100000
HIDDEN_DIM = 64
_B = 4096
_T = 200
_HALF = HIDDEN_DIM // 2


def _make_pe_np(max_loops: int, hidden_dim: int) -> np.ndarray:
    position = np.arange(0, max_loops, dtype=np.float32)[:, None]
    div_term = np.exp(
        np.arange(0, hidden_dim, 2, dtype=np.float32)
        * (-math.log(10000.0) / hidden_dim)
    )
    pe = np.zeros((max_loops, hidden_dim), dtype=np.float32)
    pe[:, 0::2] = np.sin(position * div_term)
    pe[:, 1::2] = np.cos(position * div_term)
    return pe


_PE_T = np.ascontiguousarray(_make_pe_np(MAX_LOOPS, HIDDEN_DIM).T)

_A_BLK = 8192


def _pack_body(t_ref, p_ref, o_ref):
    fused = t_ref[...] + p_ref[...]
    lo = jax.lax.bitcast_convert_type(
        fused[:_HALF, :].astype(jnp.bfloat16), jnp.uint16
    ).astype(jnp.uint32)
    hi = jax.lax.bitcast_convert_type(
        fused[_HALF:, :].astype(jnp.bfloat16), jnp.uint16
    ).astype(jnp.uint32)
    o_ref[...] = (lo | (hi << 16)).astype(jnp.int32)


def _pack_table_t(table_t):
    grid = (MAX_LOOPS + _A_BLK - 1) // _A_BLK
    return pl.pallas_call(
        _pack_body,
        grid=(grid,),
        in_specs=[
            pl.BlockSpec((HIDDEN_DIM, _A_BLK), lambda i: (0, i)),
            pl.BlockSpec((HIDDEN_DIM, _A_BLK), lambda i: (0, i)),
        ],
        out_specs=pl.BlockSpec((_HALF, _A_BLK), lambda i: (0, i)),
        out_shape=jax.ShapeDtypeStruct((_HALF, MAX_LOOPS), jnp.int32),
    )(table_t, _PE_T)


_L = 16


def _gather_t(packed, idx_t):
    mesh = plsc.VectorSubcoreMesh(core_axis_name="c", subcore_axis_name="s")

    @functools.partial(
        pl.kernel,
        out_type=jax.ShapeDtypeStruct((_T, HIDDEN_DIM, _B), jnp.float32),
        mesh=mesh,
        scratch_types=[
            pltpu.VMEM((MAX_LOOPS,), jnp.int32),
            pltpu.VMEM((2, _B), jnp.int32),
            pltpu.VMEM((2, 2, _B), jnp.float32),
            pltpu.SemaphoreType.DMA((2,)),
            pltpu.SemaphoreType.DMA((2, 2)),
        ],
        compiler_params=pltpu.CompilerParams(
            use_tc_tiling_on_sc=True, needs_layout_passes=False
        ),
    )
    def k(pack_hbm, idx_hbm, out_hbm, frow, idxv, obuf, sin, sout):
        wid = lax.axis_index("s") * 2 + lax.axis_index("c")
        pltpu.sync_copy(pack_hbm.at[wid], frow)
        pltpu.async_copy(idx_hbm.at[0], idxv.at[0], sin.at[0])

        def tbody(t, carry):
            cur = lax.rem(t, 2)
            nxt = 1 - cur

            @pl.when(t + 1 < _T)
            def _():
                pltpu.async_copy(idx_hbm.at[t + 1], idxv.at[nxt], sin.at[nxt])

            pltpu.make_async_copy(idx_hbm.at[t], idxv.at[cur], sin.at[cur]).wait()

            @pl.when(t >= 2)
            def _():
                pltpu.make_async_copy(
                    obuf.at[cur, 0], out_hbm.at[t - 2, wid], sout.at[cur, 0]
                ).wait()
                pltpu.make_async_copy(
                    obuf.at[cur, 1], out_hbm.at[t - 2, wid + _HALF], sout.at[cur, 1]
                ).wait()

            @plsc.parallel_loop(0, _B // _L, unroll=8)
            def _(g):
                o = g * _L
                vi = idxv[cur, pl.ds(o, _L)]
                v = plsc.load_gather(frow, [vi])
                obuf[cur, 0, pl.ds(o, _L)] = plsc.bitcast(v << 16, jnp.float32)
                obuf[cur, 1, pl.ds(o, _L)] = plsc.bitcast(
                    v & jnp.int32(-65536), jnp.float32
                )

            pltpu.async_copy(obuf.at[cur, 0], out_hbm.at[t, wid], sout.at[cur, 0])
            pltpu.async_copy(
                obuf.at[cur, 1], out_hbm.at[t, wid + _HALF], sout.at[cur, 1]
            )
            return carry

        lax.fori_loop(0, _T, tbody, 0)
        for tt in (_T - 2, _T - 1):
            pltpu.make_async_copy(
                obuf.at[tt % 2, 0], out_hbm.at[tt, wid], sout.at[tt % 2, 0]
            ).wait()
            pltpu.make_async_copy(
                obuf.at[tt % 2, 1], out_hbm.at[tt, wid + _HALF], sout.at[tt % 2, 1]
            ).wait()

    return k(packed, idx_t)


def kernel(loop_idx, embedding_table):
    idx_t = jnp.minimum(loop_idx, MAX_LOOPS - 1).T
    table_t = embedding_table.T
    packed = _pack_table_t(table_t)
    out_t = _gather_t(packed, idx_t)
    return jnp.transpose(out_t, (2, 0, 1))


# speedup vs baseline: 4.3500x; 1.0110x over previous
import functools
import math

import jax
import jax.numpy as jnp
import numpy as np
from jax import lax
from jax.experimental import pallas as pl
from jax.experimental.pallas import tpu as pltpu
from jax.experimental.pallas import tpu_sc as plsc

MAX_LOOPS = 100000
HIDDEN_DIM = 64
_B = 4096
_T = 200
_HALF = HIDDEN_DIM // 2


def _make_pe_np(max_loops: int, hidden_dim: int) -> np.ndarray:
    position = np.arange(0, max_loops, dtype=np.float32)[:, None]
    div_term = np.exp(
        np.arange(0, hidden_dim, 2, dtype=np.float32)
        * (-math.log(10000.0) / hidden_dim)
    )
    pe = np.zeros((max_loops, hidden_dim), dtype=np.float32)
    pe[:, 0::2] = np.sin(position * div_term)
    pe[:, 1::2] = np.cos(position * div_term)
    return pe


_PE_T = np.ascontiguousarray(_make_pe_np(MAX_LOOPS, HIDDEN_DIM).T)

_A_BLK = 8192


def _pack_body(t_ref, p_ref, o_ref):
    fused = t_ref[...] + p_ref[...]
    lo = jax.lax.bitcast_convert_type(
        fused[:_HALF, :].astype(jnp.bfloat16), jnp.uint16
    ).astype(jnp.uint32)
    hi = jax.lax.bitcast_convert_type(
        fused[_HALF:, :].astype(jnp.bfloat16), jnp.uint16
    ).astype(jnp.uint32)
    o_ref[...] = (lo | (hi << 16)).astype(jnp.int32)


def _pack_table_t(table_t):
    grid = (MAX_LOOPS + _A_BLK - 1) // _A_BLK
    return pl.pallas_call(
        _pack_body,
        grid=(grid,),
        in_specs=[
            pl.BlockSpec((HIDDEN_DIM, _A_BLK), lambda i: (0, i)),
            pl.BlockSpec((HIDDEN_DIM, _A_BLK), lambda i: (0, i)),
        ],
        out_specs=pl.BlockSpec((_HALF, _A_BLK), lambda i: (0, i)),
        out_shape=jax.ShapeDtypeStruct((_HALF, MAX_LOOPS), jnp.int32),
    )(table_t, _PE_T)


_L = 16


def _gather_t(packed, idx_t):
    mesh = plsc.VectorSubcoreMesh(core_axis_name="c", subcore_axis_name="s")

    @functools.partial(
        pl.kernel,
        out_type=jax.ShapeDtypeStruct((_T, HIDDEN_DIM, _B), jnp.float32),
        mesh=mesh,
        scratch_types=[
            pltpu.VMEM((MAX_LOOPS,), jnp.int32),
            pltpu.VMEM((2, _B), jnp.int32),
            pltpu.VMEM((2, 2, _B), jnp.float32),
            pltpu.SemaphoreType.DMA((2,)),
            pltpu.SemaphoreType.DMA((2, 2)),
        ],
        compiler_params=pltpu.CompilerParams(
            use_tc_tiling_on_sc=True, needs_layout_passes=False
        ),
    )
    def k(pack_hbm, idx_hbm, out_hbm, frow, idxv, obuf, sin, sout):
        wid = lax.axis_index("s") * 2 + lax.axis_index("c")
        pltpu.sync_copy(pack_hbm.at[wid], frow)
        pltpu.async_copy(idx_hbm.at[0], idxv.at[0], sin.at[0])

        def tbody(t, carry):
            cur = lax.rem(t, 2)
            nxt = 1 - cur

            @pl.when(t + 1 < _T)
            def _():
                pltpu.async_copy(idx_hbm.at[t + 1], idxv.at[nxt], sin.at[nxt])

            pltpu.make_async_copy(idx_hbm.at[t], idxv.at[cur], sin.at[cur]).wait()

            @pl.when(t >= 2)
            def _():
                pltpu.make_async_copy(
                    obuf.at[cur, 0], out_hbm.at[t - 2, wid], sout.at[cur, 0]
                ).wait()
                pltpu.make_async_copy(
                    obuf.at[cur, 1], out_hbm.at[t - 2, wid + _HALF], sout.at[cur, 1]
                ).wait()

            @plsc.parallel_loop(0, _B // _L, unroll=16)
            def _(g):
                o = g * _L
                vi = jnp.minimum(idxv[cur, pl.ds(o, _L)], MAX_LOOPS - 1)
                v = plsc.load_gather(frow, [vi])
                obuf[cur, 0, pl.ds(o, _L)] = plsc.bitcast(v << 16, jnp.float32)
                obuf[cur, 1, pl.ds(o, _L)] = plsc.bitcast(
                    v & jnp.int32(-65536), jnp.float32
                )

            pltpu.async_copy(obuf.at[cur, 0], out_hbm.at[t, wid], sout.at[cur, 0])
            pltpu.async_copy(
                obuf.at[cur, 1], out_hbm.at[t, wid + _HALF], sout.at[cur, 1]
            )
            return carry

        lax.fori_loop(0, _T, tbody, 0)
        for tt in (_T - 2, _T - 1):
            pltpu.make_async_copy(
                obuf.at[tt % 2, 0], out_hbm.at[tt, wid], sout.at[tt % 2, 0]
            ).wait()
            pltpu.make_async_copy(
                obuf.at[tt % 2, 1], out_hbm.at[tt, wid + _HALF], sout.at[tt % 2, 1]
            ).wait()

    return k(packed, idx_t)


def kernel(loop_idx, embedding_table):
    idx_t = loop_idx.T

    table_t = embedding_table.T
    packed = _pack_table_t(table_t)
    out_t = _gather_t(packed, idx_t)
    return jnp.transpose(out_t, (2, 0, 1))


# speedup vs baseline: 4.3886x; 1.0089x over previous
import functools
import math

import jax
import jax.numpy as jnp
import numpy as np
from jax import lax
from jax.experimental import pallas as pl
from jax.experimental.pallas import tpu as pltpu
from jax.experimental.pallas import tpu_sc as plsc

MAX_LOOPS = 100000
HIDDEN_DIM = 64
_B = 4096
_T = 200
_HALF = HIDDEN_DIM // 2


def _make_pe_np(max_loops: int, hidden_dim: int) -> np.ndarray:
    position = np.arange(0, max_loops, dtype=np.float32)[:, None]
    div_term = np.exp(
        np.arange(0, hidden_dim, 2, dtype=np.float32)
        * (-math.log(10000.0) / hidden_dim)
    )
    pe = np.zeros((max_loops, hidden_dim), dtype=np.float32)
    pe[:, 0::2] = np.sin(position * div_term)
    pe[:, 1::2] = np.cos(position * div_term)
    return pe


_PE_T = np.ascontiguousarray(_make_pe_np(MAX_LOOPS, HIDDEN_DIM).T).astype(
    jnp.bfloat16
)

_A_BLK = 8192


def _pack_body(t_ref, p_ref, o_ref):
    fused = t_ref[...] + p_ref[...].astype(jnp.float32)
    lo = jax.lax.bitcast_convert_type(
        fused[:_HALF, :].astype(jnp.bfloat16), jnp.uint16
    ).astype(jnp.uint32)
    hi = jax.lax.bitcast_convert_type(
        fused[_HALF:, :].astype(jnp.bfloat16), jnp.uint16
    ).astype(jnp.uint32)
    o_ref[...] = (lo | (hi << 16)).astype(jnp.int32)


def _pack_table_t(table_t):
    grid = (MAX_LOOPS + _A_BLK - 1) // _A_BLK
    return pl.pallas_call(
        _pack_body,
        grid=(grid,),
        in_specs=[
            pl.BlockSpec((HIDDEN_DIM, _A_BLK), lambda i: (0, i)),
            pl.BlockSpec((HIDDEN_DIM, _A_BLK), lambda i: (0, i)),
        ],
        out_specs=pl.BlockSpec((_HALF, _A_BLK), lambda i: (0, i)),
        out_shape=jax.ShapeDtypeStruct((_HALF, MAX_LOOPS), jnp.int32),
    )(table_t, _PE_T)


_L = 16


def _gather_t(packed, idx_t):
    mesh = plsc.VectorSubcoreMesh(core_axis_name="c", subcore_axis_name="s")

    @functools.partial(
        pl.kernel,
        out_type=jax.ShapeDtypeStruct((_T, HIDDEN_DIM, _B), jnp.float32),
        mesh=mesh,
        scratch_types=[
            pltpu.VMEM((MAX_LOOPS,), jnp.int32),
            pltpu.VMEM((2, _B), jnp.int32),
            pltpu.VMEM((2, 2, _B), jnp.float32),
            pltpu.SemaphoreType.DMA((2,)),
            pltpu.SemaphoreType.DMA((2, 2)),
        ],
        compiler_params=pltpu.CompilerParams(
            use_tc_tiling_on_sc=True, needs_layout_passes=False
        ),
    )
    def k(pack_hbm, idx_hbm, out_hbm, frow, idxv, obuf, sin, sout):
        wid = lax.axis_index("s") * 2 + lax.axis_index("c")
        pltpu.sync_copy(pack_hbm.at[wid], frow)
        pltpu.async_copy(idx_hbm.at[0], idxv.at[0], sin.at[0])

        def tbody(t, carry):
            cur = lax.rem(t, 2)
            nxt = 1 - cur

            @pl.when(t + 1 < _T)
            def _():
                pltpu.async_copy(idx_hbm.at[t + 1], idxv.at[nxt], sin.at[nxt])

            pltpu.make_async_copy(idx_hbm.at[t], idxv.at[cur], sin.at[cur]).wait()

            @pl.when(t >= 2)
            def _():
                pltpu.make_async_copy(
                    obuf.at[cur, 0], out_hbm.at[t - 2, wid], sout.at[cur, 0]
                ).wait()
                pltpu.make_async_copy(
                    obuf.at[cur, 1], out_hbm.at[t - 2, wid + _HALF], sout.at[cur, 1]
                ).wait()

            @plsc.parallel_loop(0, _B // _L, unroll=16)
            def _(g):
                o = g * _L
                vi = jnp.minimum(idxv[cur, pl.ds(o, _L)], MAX_LOOPS - 1)
                v = plsc.load_gather(frow, [vi])
                obuf[cur, 0, pl.ds(o, _L)] = plsc.bitcast(v << 16, jnp.float32)
                obuf[cur, 1, pl.ds(o, _L)] = plsc.bitcast(
                    v & jnp.int32(-65536), jnp.float32
                )

            pltpu.async_copy(obuf.at[cur, 0], out_hbm.at[t, wid], sout.at[cur, 0])
            pltpu.async_copy(
                obuf.at[cur, 1], out_hbm.at[t, wid + _HALF], sout.at[cur, 1]
            )
            return carry

        lax.fori_loop(0, _T, tbody, 0)
        for tt in (_T - 2, _T - 1):
            pltpu.make_async_copy(
                obuf.at[tt % 2, 0], out_hbm.at[tt, wid], sout.at[tt % 2, 0]
            ).wait()
            pltpu.make_async_copy(
                obuf.at[tt % 2, 1], out_hbm.at[tt, wid + _HALF], sout.at[tt % 2, 1]
            ).wait()

    return k(packed, idx_t)


def kernel(loop_idx, embedding_table):
    idx_t = loop_idx.T

    table_t = embedding_table.T
    packed = _pack_table_t(table_t)
    out_t = _gather_t(packed, idx_t)
    return jnp.transpose(out_t, (2, 0, 1))
